# baseline (speedup 1.0000x reference)
import jax
import jax.numpy as jnp
from jax import lax
from jax.experimental import pallas as pl
from jax.experimental.pallas import tpu as pltpu
from jax.experimental.pallas import tpu_sc as plsc

NUM_HARMONICS = 64
NUM_ENTRIES = 100000
BATCH = 16384

NC, NS, L = 2, 16, 16
NW = NC * NS
BPW = BATCH // NW
CHUNK = 128
NCHUNK = BPW // CHUNK
FUSED = 2 * NUM_HARMONICS
NFUSED = NUM_ENTRIES // 2

FPW = 1568
RCHUNK = 128
NRCHUNK = 13


def _repack_body(table_hbm, fused_hbm, in_v, out_v, in_sems, out_sems):
    wid = lax.axis_index("s") * NC + lax.axis_index("c")
    start = jnp.minimum(wid * FPW, NFUSED - FPW)

    def fire(k):
        s = k & 1
        off = min(k * RCHUNK, FPW - RCHUNK)
        return pltpu.async_copy(
            table_hbm.at[pl.ds((start + off) * 2, 2 * RCHUNK)],
            in_v.at[s], in_sems[s]), off

    inflight = {0: fire(0)}
    out_inflight = {}
    for k in range(NRCHUNK):
        if k + 1 < NRCHUNK:
            inflight[k + 1] = fire(k + 1)
        ck, off = inflight.pop(k)
        ck.wait()
        s = k & 1
        if k - 2 in out_inflight:
            out_inflight.pop(k - 2).wait()

        def row_body(f, carry, s=s):
            for h in range(2):
                for c in range(NUM_HARMONICS // L):
                    out_v[s, f, pl.ds(h * NUM_HARMONICS + c * L, L)] = (
                        in_v[s, 2 * f + h, pl.ds(c * L, L)])
            return carry

        lax.fori_loop(0, RCHUNK, row_body, 0)
        out_inflight[k] = pltpu.async_copy(
            out_v.at[s], fused_hbm.at[pl.ds(start + off, RCHUNK)],
            out_sems[s])

    for k in sorted(out_inflight):
        out_inflight[k].wait()


def _body(f_lo_hbm, f_hi_hbm, po_hbm, qo_hbm, alpha_hbm, table_hbm, out_hbm,
          f_lo_v, f_hi_v, po_v, qo_v, alpha_v,
          buf_a, buf_b, out_c, sems, out_sems):
    wid = lax.axis_index("s") * NC + lax.axis_index("c")
    base = wid * BPW

    pltpu.sync_copy(f_lo_hbm.at[pl.ds(base, BPW)], f_lo_v)
    pltpu.sync_copy(f_hi_hbm.at[pl.ds(base, BPW)], f_hi_v)
    pltpu.sync_copy(po_hbm.at[pl.ds(base, BPW)], po_v)
    pltpu.sync_copy(qo_hbm.at[pl.ds(base, BPW)], qo_v)
    pltpu.sync_copy(alpha_hbm.at[pl.ds(base, BPW)], alpha_v)

    def fire(j):
        s = j & 1
        sl = pl.ds(j * CHUNK, CHUNK)
        return (pltpu.async_copy(table_hbm.at[f_lo_v.at[sl]],
                                 buf_a.at[s], sems[2 * s]),
                pltpu.async_copy(table_hbm.at[f_hi_v.at[sl]],
                                 buf_b.at[s], sems[2 * s + 1]))

    inflight = {0: fire(0)}
    out_inflight = {}
    for j in range(NCHUNK):
        if j + 1 < NCHUNK:
            inflight[j + 1] = fire(j + 1)
        ca, cb = inflight.pop(j)
        ca.wait()
        cb.wait()
        s = j & 1
        if j - 2 in out_inflight:
            out_inflight.pop(j - 2).wait()

        def group_body(g, carry, s=s):
            gb = pl.ds(j * CHUNK + g * L, L)
            av = alpha_v[gb]
            pv = po_v[gb]
            qv = qo_v[gb]
            for k in range(L):
                a = av[k]
                p = pv[k]
                q = qv[k]
                r = g * L + k
                for c in range(NUM_HARMONICS // L):
                    x = buf_a[s, r, pl.ds(p + c * L, L)]
                    y = buf_b[s, r, pl.ds(q + c * L, L)]
                    out_c[s, r, pl.ds(c * L, L)] = x + a * (y - x)
            return carry

        lax.fori_loop(0, CHUNK // L, group_body, 0)
        out_inflight[j] = pltpu.async_copy(
            out_c.at[s], out_hbm.at[pl.ds(base + j * CHUNK, CHUNK)],
            out_sems[s])

    for j in sorted(out_inflight):
        out_inflight[j].wait()


_SC_MESH = dict(core_axis_name="c", subcore_axis_name="s",
                num_cores=NC, num_subcores=NS)
_SC_PARAMS = dict(use_tc_tiling_on_sc=True, needs_layout_passes=False)


@jax.jit
def _run(dial_2d, table):
    idx_f = dial_2d[:, 0] * float(NUM_ENTRIES - 1)
    lo = jnp.clip(idx_f.astype(jnp.int32), 0, NUM_ENTRIES - 2)
    alpha = idx_f - lo.astype(jnp.float32)
    hi = lo + 1
    f_lo = lax.shift_right_logical(lo, 1)
    f_hi = lax.shift_right_logical(hi, 1)
    po = (lo & 1) * NUM_HARMONICS
    qo = (hi & 1) * NUM_HARMONICS

    repack = pl.kernel(
        _repack_body,
        out_type=jax.ShapeDtypeStruct((NFUSED, FUSED), jnp.float32),
        mesh=plsc.VectorSubcoreMesh(**_SC_MESH),
        compiler_params=pltpu.CompilerParams(**_SC_PARAMS),
        scratch_types=[
            pltpu.VMEM((2, 2 * RCHUNK, NUM_HARMONICS), jnp.float32),
            pltpu.VMEM((2, RCHUNK, FUSED), jnp.float32),
            [pltpu.SemaphoreType.DMA] * 2,
            [pltpu.SemaphoreType.DMA] * 2,
        ],
    )
    fused = repack(table)

    mapper = pl.kernel(
        _body,
        out_type=jax.ShapeDtypeStruct((BATCH, NUM_HARMONICS), jnp.float32),
        mesh=plsc.VectorSubcoreMesh(**_SC_MESH),
        compiler_params=pltpu.CompilerParams(**_SC_PARAMS),
        scratch_types=[
            pltpu.VMEM((BPW,), jnp.int32),
            pltpu.VMEM((BPW,), jnp.int32),
            pltpu.VMEM((BPW,), jnp.int32),
            pltpu.VMEM((BPW,), jnp.int32),
            pltpu.VMEM((BPW,), jnp.float32),
            pltpu.VMEM((2, CHUNK, FUSED), jnp.float32),
            pltpu.VMEM((2, CHUNK, FUSED), jnp.float32),
            pltpu.VMEM((2, CHUNK, NUM_HARMONICS), jnp.float32),
            [pltpu.SemaphoreType.DMA] * 4,
            [pltpu.SemaphoreType.DMA] * 2,
        ],
    )
    return mapper(f_lo, f_hi, po, qo, alpha, fused)


def kernel(wave_dial_normalized, table):
    return _run(wave_dial_normalized, table)


# speedup vs baseline: 1.1433x; 1.1433x over previous
import jax
import jax.numpy as jnp
from jax import lax
from jax.experimental import pallas as pl
from jax.experimental.pallas import tpu as pltpu
from jax.experimental.pallas import tpu_sc as plsc

NUM_HARMONICS = 64
NUM_ENTRIES = 100000
BATCH = 16384

NC, NS, L = 2, 16, 16
NW = NC * NS
RPW = NUM_HARMONICS // NW
ECHUNK = 2048
NECHUNK = BATCH // ECHUNK


def _body(lo_hbm, hi_hbm, alpha_hbm, tab_t_hbm, out_t_hbm,
          row_v, lo_v, hi_v, alpha_v, out_v, row_sem, out_sems):
    wid = lax.axis_index("s") * NC + lax.axis_index("c")

    zeros = jnp.zeros((L,), jnp.int32)
    out_inflight = {}
    for r in range(RPW):
        c = wid * RPW + r
        pltpu.async_copy(tab_t_hbm.at[pl.ds(c, 1)], row_v, row_sem).wait()
        for k in range(NECHUNK):
            ek = pl.ds(k * ECHUNK, ECHUNK)
            pltpu.sync_copy(lo_hbm.at[ek], lo_v)
            pltpu.sync_copy(hi_hbm.at[ek], hi_v)
            pltpu.sync_copy(alpha_hbm.at[ek], alpha_v)
            s = k & 1
            key = (r, k - 2)
            if key in out_inflight:
                out_inflight.pop(key).wait()

            def group_body(g, carry, s=s):
                gb = pl.ds(g * L, L)
                lov = lo_v[gb]
                hiv = hi_v[gb]
                av = alpha_v[gb]
                x = plsc.load_gather(row_v, [zeros, lov])
                y = plsc.load_gather(row_v, [zeros, hiv])
                out_v[s, 0, gb] = x + av * (y - x)
                return carry

            lax.fori_loop(0, ECHUNK // L, group_body, 0)
            out_inflight[(r, k)] = pltpu.async_copy(
                out_v.at[s],
                out_t_hbm.at[pl.ds(c, 1), pl.ds(k * ECHUNK, ECHUNK)],
                out_sems[s])

    for key in sorted(out_inflight):
        out_inflight[key].wait()


@jax.jit
def _run(dial_2d, table):
    idx_f = dial_2d[:, 0] * float(NUM_ENTRIES - 1)
    lo = jnp.clip(idx_f.astype(jnp.int32), 0, NUM_ENTRIES - 2)
    alpha = idx_f - lo.astype(jnp.float32)
    hi = lo + 1

    mapper = pl.kernel(
        _body,
        out_type=jax.ShapeDtypeStruct((NUM_HARMONICS, BATCH), jnp.float32),
        mesh=plsc.VectorSubcoreMesh(
            core_axis_name="c", subcore_axis_name="s",
            num_cores=NC, num_subcores=NS),
        compiler_params=pltpu.CompilerParams(
            use_tc_tiling_on_sc=False, needs_layout_passes=False),
        scratch_types=[
            pltpu.VMEM((1, NUM_ENTRIES), jnp.float32),
            pltpu.VMEM((ECHUNK,), jnp.int32),
            pltpu.VMEM((ECHUNK,), jnp.int32),
            pltpu.VMEM((ECHUNK,), jnp.float32),
            pltpu.VMEM((2, 1, ECHUNK), jnp.float32),
            pltpu.SemaphoreType.DMA,
            [pltpu.SemaphoreType.DMA] * 2,
        ],
    )
    out_t = mapper(lo, hi, alpha, table.T)
    return out_t.T


def kernel(wave_dial_normalized, table):
    return _run(wave_dial_normalized, table)


# speedup vs baseline: 1.5773x; 1.3796x over previous
import jax
import jax.numpy as jnp
from jax import lax
from jax.experimental import pallas as pl
from jax.experimental.pallas import tpu as pltpu
from jax.experimental.pallas import tpu_sc as plsc

NUM_HARMONICS = 64
NUM_ENTRIES = 100000
BATCH = 16384

NC, NS, L = 2, 16, 16
NW = NC * NS
RPW = NUM_HARMONICS // NW
ECHUNK = 2048
NECHUNK = BATCH // ECHUNK


def _body(idxf_hbm, tab_t_hbm, out_t_hbm,
          row_v, idxf_v, out_v, row_sem, idx_sem, out_sems):
    wid = lax.axis_index("s") * NC + lax.axis_index("c")

    zeros = jnp.zeros((L,), jnp.int32)
    idx_cp = pltpu.async_copy(idxf_hbm, idxf_v, idx_sem)
    out_inflight = {}
    for r in range(RPW):
        c = wid * RPW + r
        pltpu.async_copy(tab_t_hbm.at[pl.ds(c, 1)], row_v, row_sem).wait()
        if r == 0:
            idx_cp.wait()
        for k in range(NECHUNK):
            s = k & 1
            key = (r, k - 2)
            if key in out_inflight:
                out_inflight.pop(key).wait()

            def group_body(g, carry, k=k, s=s):
                gb = pl.ds(k * ECHUNK + g * L, L)
                f = idxf_v[gb]
                lov = f.astype(jnp.int32)
                lov = jnp.minimum(lov, NUM_ENTRIES - 2)
                av = f - lov.astype(jnp.float32)
                x = plsc.load_gather(row_v, [zeros, lov])
                y = plsc.load_gather(row_v, [zeros, lov + 1])
                out_v[s, 0, pl.ds(g * L, L)] = x + av * (y - x)
                return carry

            lax.fori_loop(0, ECHUNK // L, group_body, 0)
            out_inflight[(r, k)] = pltpu.async_copy(
                out_v.at[s],
                out_t_hbm.at[pl.ds(c, 1), pl.ds(k * ECHUNK, ECHUNK)],
                out_sems[s])

    for key in sorted(out_inflight):
        out_inflight[key].wait()


@jax.jit
def _run(dial_2d, table):
    idx_f = dial_2d[:, 0] * float(NUM_ENTRIES - 1)

    mapper = pl.kernel(
        _body,
        out_type=jax.ShapeDtypeStruct((NUM_HARMONICS, BATCH), jnp.float32),
        mesh=plsc.VectorSubcoreMesh(
            core_axis_name="c", subcore_axis_name="s",
            num_cores=NC, num_subcores=NS),
        compiler_params=pltpu.CompilerParams(
            use_tc_tiling_on_sc=False, needs_layout_passes=False),
        scratch_types=[
            pltpu.VMEM((1, NUM_ENTRIES), jnp.float32),
            pltpu.VMEM((BATCH,), jnp.float32),
            pltpu.VMEM((2, 1, ECHUNK), jnp.float32),
            pltpu.SemaphoreType.DMA,
            pltpu.SemaphoreType.DMA,
            [pltpu.SemaphoreType.DMA] * 2,
        ],
    )
    out_t = mapper(idx_f, table.T)
    return out_t.T


def kernel(wave_dial_normalized, table):
    return _run(wave_dial_normalized, table)
